# Initial kernel scaffold; baseline (speedup 1.0000x reference)
import jax
import jax.numpy as jnp
from jax.experimental import pallas as pl


def kernel(input):
    raise NotImplementedError("write your pallas kernel here")


# speedup vs baseline: 23.2463x; 23.2463x over previous
import functools

import jax
import jax.numpy as jnp
from jax.experimental import pallas as pl

TOPK = 5
STRIDE = 4.0
C_BLK = 8
NEG = float("-inf")


def _body(x_ref, o_ref):
    X = x_ref[0, 0]
    rowmax = jnp.max(X, axis=2)

    i_iota = jax.lax.broadcasted_iota(jnp.int32, (C_BLK, 128), 1)
    j_iota = i_iota

    picked_i, picked_j = [], []
    scores, xs, ys = [], [], []
    for _t in range(TOPK):
        v = jnp.max(rowmax, axis=1, keepdims=True)
        i_star = jnp.min(jnp.where(rowmax == v, i_iota, 128), axis=1)

        rows = [
            x_ref[0, 0, c, pl.ds(i_star[c], 1), :]
            for c in range(C_BLK)
        ]
        row = jnp.concatenate(rows, axis=0)
        for pi, pj in zip(picked_i, picked_j):
            same = (pi[:, None] == i_star[:, None]) & (j_iota == pj[:, None])
            row = jnp.where(same, NEG, row)

        j_star = jnp.min(jnp.where(row == v, j_iota, 128), axis=1)
        second = jnp.max(
            jnp.where(j_iota == j_star[:, None], NEG, row), axis=1
        )
        rowmax = jnp.where(i_iota == i_star[:, None], second[:, None], rowmax)

        picked_i.append(i_star)
        picked_j.append(j_star)
        scores.append(v[:, 0])
        xs.append(j_star)
        ys.append(i_star)

    S = jnp.stack(scores, axis=1)
    P = jax.nn.softmax(S, axis=1)
    Xc = jnp.stack(xs, axis=1).astype(jnp.float32)
    Yc = jnp.stack(ys, axis=1).astype(jnp.float32)
    ox = jnp.sum(Xc * P, axis=1)
    oy = jnp.sum(Yc * P, axis=1)
    o_ref[0] = STRIDE * jnp.stack([ox, oy], axis=1)


@functools.partial(jax.jit, static_argnames=("interpret",))
def kernel(input, interpret=False):
    N, _two, C, H, W = input.shape
    grid = (N, C // C_BLK)
    return pl.pallas_call(
        _body,
        grid=grid,
        in_specs=[
            pl.BlockSpec(
                (1, 1, C_BLK, H, W), lambda n, c: (n, 1, c, 0, 0)
            )
        ],
        out_specs=pl.BlockSpec((1, C_BLK, 2), lambda n, c: (n, c, 0)),
        out_shape=jax.ShapeDtypeStruct((N, C, 2), jnp.float32),
        interpret=interpret,
    )(input)


# speedup vs baseline: 126.2179x; 5.4296x over previous
import functools

import jax
import jax.numpy as jnp
from jax.experimental import pallas as pl

TOPK = 5
STRIDE = 4.0
C_BLK = 96
NEG = float("-inf")


def _body(x_ref, o_ref):
    X = x_ref[0, 0]
    rowmax = jnp.max(X, axis=2)

    i_iota = jax.lax.broadcasted_iota(jnp.int32, (C_BLK, 128), 1)
    j_iota = i_iota

    picked_i, picked_j = [], []
    scores, xs, ys = [], [], []
    for _t in range(TOPK):
        v = jnp.max(rowmax, axis=1, keepdims=True)
        i_star = jnp.min(jnp.where(rowmax == v, i_iota, 128), axis=1)

        rows = [
            x_ref[0, 0, c, pl.ds(i_star[c], 1), :]
            for c in range(C_BLK)
        ]
        row = jnp.concatenate(rows, axis=0)
        for pi, pj in zip(picked_i, picked_j):
            same = (pi[:, None] == i_star[:, None]) & (j_iota == pj[:, None])
            row = jnp.where(same, NEG, row)

        j_star = jnp.min(jnp.where(row == v, j_iota, 128), axis=1)
        second = jnp.max(
            jnp.where(j_iota == j_star[:, None], NEG, row), axis=1
        )
        rowmax = jnp.where(i_iota == i_star[:, None], second[:, None], rowmax)

        picked_i.append(i_star)
        picked_j.append(j_star)
        scores.append(v[:, 0])
        xs.append(j_star)
        ys.append(i_star)

    S = jnp.stack(scores, axis=1)
    P = jax.nn.softmax(S, axis=1)
    Xc = jnp.stack(xs, axis=1).astype(jnp.float32)
    Yc = jnp.stack(ys, axis=1).astype(jnp.float32)
    ox = jnp.sum(Xc * P, axis=1)
    oy = jnp.sum(Yc * P, axis=1)
    o_ref[0] = STRIDE * jnp.stack([ox, oy], axis=1)


@functools.partial(jax.jit, static_argnames=("interpret",))
def kernel(input, interpret=False):
    N, _two, C, H, W = input.shape
    grid = (N, C // C_BLK)
    return pl.pallas_call(
        _body,
        grid=grid,
        in_specs=[
            pl.BlockSpec(
                (1, 1, C_BLK, H, W), lambda n, c: (n, 1, c, 0, 0)
            )
        ],
        out_specs=pl.BlockSpec((1, C_BLK, 2), lambda n, c: (n, c, 0)),
        out_shape=jax.ShapeDtypeStruct((N, C, 2), jnp.float32),
        interpret=interpret,
    )(input)


# speedup vs baseline: 198.2451x; 1.5707x over previous
import functools

import jax
import jax.numpy as jnp
from jax.experimental import pallas as pl
from jax.experimental.pallas import tpu as pltpu

TOPK = 5
STRIDE = 4.0
N_BLK = 2
C_ALL = 96
R_BLK = N_BLK * C_ALL
NEG = float("-inf")


def _body(x_ref, o_ref, ivmem_ref, ismem_ref, sems):
    lane = jax.lax.broadcasted_iota(jnp.int32, (R_BLK, 128), 1)

    X = x_ref[:, 0].reshape(R_BLK, 128, 128)
    rowmax = jnp.max(jnp.transpose(X, (0, 2, 1)), axis=1)

    rm = rowmax
    row_ids, copies = [], []
    for t in range(TOPK):
        v = jnp.max(rm, axis=1, keepdims=True)
        i_t = jnp.min(jnp.where(rm == v, lane, 128), axis=1)
        rm = jnp.where(lane == i_t[:, None], NEG, rm)
        row_ids.append(i_t)
        ivmem_ref[pl.ds(t, 1), pl.ds(0, R_BLK)] = i_t[None, :]
        copy = pltpu.make_async_copy(
            ivmem_ref.at[pl.ds(t, 1)], ismem_ref.at[pl.ds(t, 1)], sems.at[t]
        )
        copy.start()
        copies.append(copy)

    parts, fparts = [], []
    for t, i_t in enumerate(row_ids):
        copies[t].wait()
        rows = jnp.concatenate(
            [
                x_ref[c // C_ALL, 0, c % C_ALL, pl.ds(ismem_ref[t, c], 1), :]
                for c in range(R_BLK)
            ],
            axis=0,
        )
        parts.append(rows)
        fparts.append(i_t[:, None] * 128 + lane)
    R = jnp.concatenate(parts, axis=1)
    F = jnp.concatenate(fparts, axis=1)

    scores, fidx = [], []
    for _t in range(TOPK):
        v = jnp.max(R, axis=1, keepdims=True)
        f = jnp.min(jnp.where(R == v, F, 1 << 20), axis=1, keepdims=True)
        R = jnp.where(F == f, NEG, R)
        scores.append(v[:, 0])
        fidx.append(f[:, 0])

    S = jnp.stack(scores, axis=1)
    P = jax.nn.softmax(S, axis=1)
    FI = jnp.stack(fidx, axis=1)
    Xc = (FI % 128).astype(jnp.float32)
    Yc = (FI // 128).astype(jnp.float32)
    ox = jnp.sum(Xc * P, axis=1)
    oy = jnp.sum(Yc * P, axis=1)
    out = STRIDE * jnp.stack([ox, oy], axis=1)
    o_ref[...] = out.reshape(N_BLK, C_ALL, 2)


@functools.partial(jax.jit, static_argnames=("interpret",))
def kernel(input, interpret=False):
    N, _two, C, H, W = input.shape
    grid = (N // N_BLK,)
    return pl.pallas_call(
        _body,
        grid=grid,
        in_specs=[
            pl.BlockSpec(
                (N_BLK, 1, C, H, W), lambda n: (n, 1, 0, 0, 0)
            )
        ],
        out_specs=pl.BlockSpec((N_BLK, C, 2), lambda n: (n, 0, 0)),
        out_shape=jax.ShapeDtypeStruct((N, C, 2), jnp.float32),
        scratch_shapes=[
            pltpu.VMEM((8, 256), jnp.int32),
            pltpu.SMEM((8, 256), jnp.int32),
            pltpu.SemaphoreType.DMA((TOPK,)),
        ],
        interpret=interpret,
    )(input)


# speedup vs baseline: 215.6735x; 1.0879x over previous
import functools

import jax
import jax.numpy as jnp
from jax.experimental import pallas as pl
from jax.experimental.pallas import tpu as pltpu

TOPK = 5
STRIDE = 4.0
N_BLK = 4
C_ALL = 96
R_BLK = N_BLK * C_ALL
NEG = float("-inf")


def _body(x_ref, o_ref, ivmem_ref, ismem_ref, sems):
    lane = jax.lax.broadcasted_iota(jnp.int32, (R_BLK, 128), 1)

    X = x_ref[:, 0].reshape(R_BLK, 128, 128)
    rowmax = jnp.max(jnp.transpose(X, (0, 2, 1)), axis=1)

    rm = rowmax
    row_ids, copies = [], []
    for t in range(TOPK):
        v = jnp.max(rm, axis=1, keepdims=True)
        i_t = jnp.min(jnp.where(rm == v, lane, 128), axis=1)
        rm = jnp.where(lane == i_t[:, None], NEG, rm)
        row_ids.append(i_t)
        ivmem_ref[pl.ds(t, 1), pl.ds(0, R_BLK)] = i_t[None, :]
        copy = pltpu.make_async_copy(
            ivmem_ref.at[pl.ds(t, 1)], ismem_ref.at[pl.ds(t, 1)], sems.at[t]
        )
        copy.start()
        copies.append(copy)

    parts, fparts = [], []
    for t, i_t in enumerate(row_ids):
        copies[t].wait()
        rows = jnp.concatenate(
            [
                x_ref[c // C_ALL, 0, c % C_ALL, pl.ds(ismem_ref[t, c], 1), :]
                for c in range(R_BLK)
            ],
            axis=0,
        )
        parts.append(rows)
        fparts.append(i_t[:, None] * 128 + lane)
    R = jnp.concatenate(parts, axis=1)
    F = jnp.concatenate(fparts, axis=1)

    scores, fidx = [], []
    for _t in range(TOPK):
        v = jnp.max(R, axis=1, keepdims=True)
        f = jnp.min(jnp.where(R == v, F, 1 << 20), axis=1, keepdims=True)
        R = jnp.where(F == f, NEG, R)
        scores.append(v[:, 0])
        fidx.append(f[:, 0])

    S = jnp.stack(scores, axis=1)
    P = jax.nn.softmax(S, axis=1)
    FI = jnp.stack(fidx, axis=1)
    Xc = (FI % 128).astype(jnp.float32)
    Yc = (FI // 128).astype(jnp.float32)
    ox = jnp.sum(Xc * P, axis=1)
    oy = jnp.sum(Yc * P, axis=1)
    out = STRIDE * jnp.stack([ox, oy], axis=1)
    o_ref[...] = out.reshape(N_BLK, C_ALL, 2)


@functools.partial(jax.jit, static_argnames=("interpret",))
def kernel(input, interpret=False):
    N, _two, C, H, W = input.shape
    grid = (N // N_BLK,)
    return pl.pallas_call(
        _body,
        grid=grid,
        in_specs=[
            pl.BlockSpec(
                (N_BLK, 1, C, H, W), lambda n: (n, 1, 0, 0, 0)
            )
        ],
        out_specs=pl.BlockSpec((N_BLK, C, 2), lambda n: (n, 0, 0)),
        out_shape=jax.ShapeDtypeStruct((N, C, 2), jnp.float32),
        scratch_shapes=[
            pltpu.VMEM((8, 512), jnp.int32),
            pltpu.SMEM((8, 512), jnp.int32),
            pltpu.SemaphoreType.DMA((TOPK,)),
        ],
        interpret=interpret,
    )(input)


# speedup vs baseline: 241.5214x; 1.1198x over previous
import functools

import jax
import jax.numpy as jnp
from jax.experimental import pallas as pl
from jax.experimental.pallas import tpu as pltpu

TOPK = 5
STRIDE = 4.0
N_BLK = 4
C_ALL = 96
R_BLK = N_BLK * C_ALL
NEG = float("-inf")


def _body(x_ref, o_ref, ivmem_ref, ismem_ref, sems):
    lane = jax.lax.broadcasted_iota(jnp.int32, (R_BLK, 128), 1)

    X = x_ref[:, 0].reshape(R_BLK, 128, 128)
    rowmax = jnp.max(jnp.transpose(X, (0, 2, 1)), axis=1)

    rmT = rowmax.T
    sub = jax.lax.broadcasted_iota(jnp.int32, (128, R_BLK), 0)
    row_ids, copies = [], []
    for t in range(TOPK):
        v = jnp.max(rmT, axis=0, keepdims=True)
        i_t = jnp.min(jnp.where(rmT == v, sub, 128), axis=0)
        rmT = jnp.where(sub == i_t[None, :], NEG, rmT)
        row_ids.append(i_t)
        ivmem_ref[pl.ds(t, 1), pl.ds(0, R_BLK)] = i_t[None, :]
        copy = pltpu.make_async_copy(
            ivmem_ref.at[pl.ds(t, 1)], ismem_ref.at[pl.ds(t, 1)], sems.at[t]
        )
        copy.start()
        copies.append(copy)

    sub128 = jax.lax.broadcasted_iota(jnp.int32, (128, R_BLK), 0)
    parts, fparts = [], []
    for t, i_t in enumerate(row_ids):
        copies[t].wait()
        rows = jnp.concatenate(
            [
                x_ref[c // C_ALL, 0, c % C_ALL, pl.ds(ismem_ref[t, c], 1), :]
                for c in range(R_BLK)
            ],
            axis=0,
        )
        parts.append(rows)
        fparts.append(i_t[None, :] * 128 + sub128)
    RT = jnp.concatenate(parts, axis=1).T
    FT = jnp.concatenate(fparts, axis=0)

    scores, fidx = [], []
    for _t in range(TOPK):
        v = jnp.max(RT, axis=0, keepdims=True)
        f = jnp.min(jnp.where(RT == v, FT, 1 << 20), axis=0, keepdims=True)
        RT = jnp.where(FT == f, NEG, RT)
        scores.append(v[0])
        fidx.append(f[0])

    S = jnp.stack(scores, axis=0)
    P = jax.nn.softmax(S, axis=0)
    FI = jnp.stack(fidx, axis=0)
    Xc = (FI % 128).astype(jnp.float32)
    Yc = (FI // 128).astype(jnp.float32)
    ox = jnp.sum(Xc * P, axis=0)
    oy = jnp.sum(Yc * P, axis=0)
    out = STRIDE * jnp.stack([ox, oy], axis=0).T
    o_ref[...] = out.reshape(N_BLK, C_ALL, 2)


@functools.partial(jax.jit, static_argnames=("interpret",))
def kernel(input, interpret=False):
    N, _two, C, H, W = input.shape
    grid = (N // N_BLK,)
    return pl.pallas_call(
        _body,
        grid=grid,
        in_specs=[
            pl.BlockSpec(
                (N_BLK, 1, C, H, W), lambda n: (n, 1, 0, 0, 0)
            )
        ],
        out_specs=pl.BlockSpec((N_BLK, C, 2), lambda n: (n, 0, 0)),
        out_shape=jax.ShapeDtypeStruct((N, C, 2), jnp.float32),
        scratch_shapes=[
            pltpu.VMEM((8, 512), jnp.int32),
            pltpu.SMEM((8, 512), jnp.int32),
            pltpu.SemaphoreType.DMA((TOPK,)),
        ],
        interpret=interpret,
    )(input)
